# Initial kernel scaffold; baseline (speedup 1.0000x reference)
import jax
import jax.numpy as jnp
from jax.experimental import pallas as pl


def kernel(p1, x):
    raise NotImplementedError("write your pallas kernel here")


# speedup vs baseline: 2.0423x; 2.0423x over previous
import functools

import jax
import jax.numpy as jnp
from jax import lax
from jax.experimental import pallas as pl
from jax.experimental.pallas import tpu as pltpu

STRIDE = 4
NSAMPLE = 16


def _fps_kernel(p_ref, idx_ref, p2x_ref, p2y_ref, p2z_ref, dists_ref):
    B, S, L = p_ref.shape[1], p_ref.shape[2], p_ref.shape[3]
    M = idx_ref.shape[0]
    N = S * L

    flat_iota = (lax.broadcasted_iota(jnp.int32, (B, S, L), 1) * L
                 + lax.broadcasted_iota(jnp.int32, (B, S, L), 2))

    px = p_ref[0]
    py = p_ref[1]
    pz = p_ref[2]

    lx0 = px[:, 0:1, 0:1]
    ly0 = py[:, 0:1, 0:1]
    lz0 = pz[:, 0:1, 0:1]
    idx_ref[0:1, :] = jnp.zeros((1, B), jnp.int32)
    p2x_ref[0:1, :] = lx0.reshape(1, B)
    p2y_ref[0:1, :] = ly0.reshape(1, B)
    p2z_ref[0:1, :] = lz0.reshape(1, B)
    dists_ref[...] = jnp.full((B, S, L), 1e10, jnp.float32)

    def body(i, carry):
        lx, ly, lz = carry
        dx = px - lx
        dy = py - ly
        dz = pz - lz
        d = dx * dx + dy * dy + dz * dz
        dists = jnp.minimum(dists_ref[...], d)
        dists_ref[...] = dists
        m = jnp.max(dists, axis=(1, 2), keepdims=True)
        eq = dists == m
        pos = jnp.min(jnp.where(eq, flat_iota, jnp.int32(N)),
                      axis=(1, 2), keepdims=True)
        sel = (flat_iota == pos).astype(jnp.float32)
        nlx = jnp.sum(px * sel, axis=(1, 2), keepdims=True)
        nly = jnp.sum(py * sel, axis=(1, 2), keepdims=True)
        nlz = jnp.sum(pz * sel, axis=(1, 2), keepdims=True)
        idx_ref[pl.ds(i, 1), :] = pos[:, 0, 0].reshape(1, B)
        p2x_ref[pl.ds(i, 1), :] = nlx[:, 0, 0].reshape(1, B)
        p2y_ref[pl.ds(i, 1), :] = nly[:, 0, 0].reshape(1, B)
        p2z_ref[pl.ds(i, 1), :] = nlz[:, 0, 0].reshape(1, B)
        return (nlx, nly, nlz)

    lax.fori_loop(1, M, body, (lx0, ly0, lz0))


def _fps(p1):
    B, N, _ = p1.shape
    M = N // STRIDE
    S, L = N // 128, 128
    pt = jnp.transpose(p1, (2, 0, 1)).reshape(3, B, S, L)
    idx_t, p2x, p2y, p2z = pl.pallas_call(
        _fps_kernel,
        out_shape=(
            jax.ShapeDtypeStruct((M, B), jnp.int32),
            jax.ShapeDtypeStruct((M, B), jnp.float32),
            jax.ShapeDtypeStruct((M, B), jnp.float32),
            jax.ShapeDtypeStruct((M, B), jnp.float32),
        ),
        scratch_shapes=[pltpu.VMEM((B, S, L), jnp.float32)],
    )(pt)
    idx = idx_t.T
    p2 = jnp.stack([p2x.T, p2y.T, p2z.T], axis=-1)
    return idx, p2


def kernel(p1, x):
    B, N, C = x.shape
    idx, p2 = _fps(p1)
    d2 = (jnp.sum(p2 ** 2, axis=-1)[:, :, None]
          + jnp.sum(p1 ** 2, axis=-1)[:, None, :]
          - 2.0 * jnp.einsum('bmd,bnd->bmn', p2, p1))
    _, nn_idx = lax.top_k(-d2, NSAMPLE)
    feats = jax.vmap(lambda xb, ib: xb[ib])(x, nn_idx)
    y = jnp.mean(feats, axis=2)
    return (y, p2, idx.astype(jnp.int64))


# speedup vs baseline: 19.6045x; 9.5991x over previous
import functools

import jax
import jax.numpy as jnp
from jax import lax
from jax.experimental import pallas as pl
from jax.experimental.pallas import tpu as pltpu

STRIDE = 4
NSAMPLE = 16


def _fps_kernel(p_ref, idx_ref, p2x_ref, p2y_ref, p2z_ref, dists_ref):
    B, S, L = p_ref.shape[1], p_ref.shape[2], p_ref.shape[3]
    M = idx_ref.shape[0]
    N = S * L

    flat_iota = (lax.broadcasted_iota(jnp.int32, (B, S, L), 1) * L
                 + lax.broadcasted_iota(jnp.int32, (B, S, L), 2))

    px = p_ref[0]
    py = p_ref[1]
    pz = p_ref[2]

    lx0 = px[:, 0:1, 0:1]
    ly0 = py[:, 0:1, 0:1]
    lz0 = pz[:, 0:1, 0:1]
    idx_ref[0:1, :] = jnp.zeros((1, B), jnp.int32)
    p2x_ref[0:1, :] = lx0.reshape(1, B)
    p2y_ref[0:1, :] = ly0.reshape(1, B)
    p2z_ref[0:1, :] = lz0.reshape(1, B)
    dists_ref[...] = jnp.full((B, S, L), 1e10, jnp.float32)

    def body(i, carry):
        lx, ly, lz = carry
        dx = px - lx
        dy = py - ly
        dz = pz - lz
        d = (dx * dx + dz * dz) + dy * dy
        dists = jnp.minimum(dists_ref[...], d)
        dists_ref[...] = dists
        m = jnp.max(dists, axis=(1, 2), keepdims=True)
        eq = dists == m
        pos = jnp.min(jnp.where(eq, flat_iota, jnp.int32(N)),
                      axis=(1, 2), keepdims=True)
        sel = (flat_iota == pos).astype(jnp.float32)
        nlx = jnp.sum(px * sel, axis=(1, 2), keepdims=True)
        nly = jnp.sum(py * sel, axis=(1, 2), keepdims=True)
        nlz = jnp.sum(pz * sel, axis=(1, 2), keepdims=True)
        idx_ref[pl.ds(i, 1), :] = pos[:, 0, 0].reshape(1, B)
        p2x_ref[pl.ds(i, 1), :] = nlx[:, 0, 0].reshape(1, B)
        p2y_ref[pl.ds(i, 1), :] = nly[:, 0, 0].reshape(1, B)
        p2z_ref[pl.ds(i, 1), :] = nlz[:, 0, 0].reshape(1, B)
        return (nlx, nly, nlz)

    lax.fori_loop(1, M, body, (lx0, ly0, lz0))


def _fps(p1):
    B, N, _ = p1.shape
    M = N // STRIDE
    S, L = N // 128, 128
    pt = jnp.transpose(p1, (2, 0, 1)).reshape(3, B, S, L)
    idx_t, p2x, p2y, p2z = pl.pallas_call(
        _fps_kernel,
        out_shape=(
            jax.ShapeDtypeStruct((M, B), jnp.int32),
            jax.ShapeDtypeStruct((M, B), jnp.float32),
            jax.ShapeDtypeStruct((M, B), jnp.float32),
            jax.ShapeDtypeStruct((M, B), jnp.float32),
        ),
        scratch_shapes=[pltpu.VMEM((B, S, L), jnp.float32)],
    )(pt)
    idx = idx_t.T
    p2 = jnp.stack([p2x.T, p2y.T, p2z.T], axis=-1)
    return idx, p2


def _knn_pool_kernel(qx_ref, qy_ref, qz_ref, px_ref, py_ref, pz_ref,
                     qm_ref, pm_ref, x_ref, y_ref, d2_ref):
    TM, N = d2_ref.shape
    qx = qx_ref[0, 0].reshape(TM, 1)
    qy = qy_ref[0, 0].reshape(TM, 1)
    qz = qz_ref[0, 0].reshape(TM, 1)
    px = px_ref[0]
    py = py_ref[0]
    pz = pz_ref[0]
    sq = (qx * qx + qz * qz) + qy * qy
    sp = (px * px + pz * pz) + py * py
    g = lax.dot_general(qm_ref[0], pm_ref[0], (((1,), (1,)), ((), ())),
                        preferred_element_type=jnp.float32)
    d2_ref[...] = sq + sp - 2.0 * g

    ii = lax.broadcasted_iota(jnp.int32, (TM, N), 1)
    inf = jnp.float32(jnp.inf)
    for _ in range(NSAMPLE):
        d2 = d2_ref[...]
        mn = jnp.min(d2, axis=1, keepdims=True)
        pos = jnp.min(jnp.where(d2 == mn, ii, jnp.int32(N)),
                      axis=1, keepdims=True)
        d2_ref[...] = jnp.where(ii == pos, inf, d2)

    s = (d2_ref[...] == inf).astype(jnp.float32)
    y = lax.dot_general(s, x_ref[0], (((1,), (0,)), ((), ())),
                        preferred_element_type=jnp.float32)
    y_ref[0] = y * jnp.float32(1.0 / NSAMPLE)


def _knn_pool(p2t, p2, p1, x, TM=512):
    _, B, M = p2t.shape
    _, N, C = x.shape
    MT = M // TM
    p1t = jnp.transpose(p1, (2, 0, 1)).reshape(3, B, 1, N)
    q = p2t.reshape(3, B * MT, 1, TM)
    qm = p2.reshape(B * MT, TM, 3)
    grid = (B, MT)
    qspec = pl.BlockSpec((1, 1, TM), lambda b, m: (b * MT + m, 0, 0))
    pspec = pl.BlockSpec((1, 1, N), lambda b, m: (b, 0, 0))
    return pl.pallas_call(
        _knn_pool_kernel,
        grid=grid,
        in_specs=[qspec, qspec, qspec, pspec, pspec, pspec,
                  pl.BlockSpec((1, TM, 3), lambda b, m: (b * MT + m, 0, 0)),
                  pl.BlockSpec((1, N, 3), lambda b, m: (b, 0, 0)),
                  pl.BlockSpec((1, N, C), lambda b, m: (b, 0, 0))],
        out_specs=pl.BlockSpec((1, TM, C), lambda b, m: (b, m, 0)),
        out_shape=jax.ShapeDtypeStruct((B, M, C), jnp.float32),
        scratch_shapes=[pltpu.VMEM((TM, N), jnp.float32)],
    )(q[0], q[1], q[2], p1t[0], p1t[1], p1t[2], qm, p1, x)


def kernel(p1, x):
    B, N, C = x.shape
    idx, p2 = _fps(p1)
    p2t = jnp.transpose(p2, (2, 0, 1))
    y = _knn_pool(p2t, p2, p1, x)
    return (y, p2, idx.astype(jnp.int64))
